# baseline (speedup 1.0000x reference)
import functools

import numpy as np
import jax
import jax.numpy as jnp
from jax.experimental import pallas as pl
from jax.experimental.pallas import tpu as pltpu


def _up_sel(Wu):
    Wd = 2 * Wu
    sel = np.zeros((Wu, Wd, 2), np.float32)
    for x in range(Wu):
        for dj in range(2):
            sel[x, 2 * x + dj, dj] = 1.0
    return sel


def _band_sel(Wd):
    sel = np.zeros((Wd, Wd, 3), np.float32)
    for i in range(Wd):
        for dx in range(3):
            j = i - (dx - 1)
            if 0 <= j < Wd:
                sel[i, j, dx] = 1.0
    return sel


def _up_pair_mats(wt, Wu):
    Cin, Cout = wt.shape[0], wt.shape[1]
    Wd = 2 * Wu
    mu = jnp.einsum('xjd,cokd->kxcjo', _up_sel(Wu), wt.astype(jnp.float32))
    return mu.reshape(2, Wu * Cin, Wd * Cout)


def _band_mats(w_oihw, Wd):
    m = jnp.einsum('ijd,ockd->kicjo', _band_sel(Wd), w_oihw.astype(jnp.float32))
    Cout, Cin = w_oihw.shape[0], w_oihw.shape[1]
    return m.reshape(3, Wd * Cin, Wd * Cout)


def _dot(a, b):
    return jnp.dot(a, b, preferred_element_type=jnp.float32)


def _shift_down(x):
    return jnp.concatenate([jnp.zeros_like(x[:1]), x[:-1]], axis=0)


def _shift_up(x):
    return jnp.concatenate([x[1:], jnp.zeros_like(x[:1])], axis=0)


def _fused_kernel(fu_ref, fd_ref, mu_ref, w1u_ref, w1f_ref, w2_ref,
                  btr_ref, b1r_ref, b2r_ref, o_ref):
    bf16 = jnp.bfloat16
    fu = fu_ref[0]
    fd_e = fd_ref[0, :, 0, :]
    fd_o = fd_ref[0, :, 1, :]

    up_e = (_dot(fu, mu_ref[0]) + btr_ref[...]).astype(bf16)
    up_o = (_dot(fu, mu_ref[1]) + btr_ref[...]).astype(bf16)

    uo_m1, fo_m1 = _shift_down(up_o), _shift_down(fd_o)
    ue_p1, fe_p1 = _shift_up(up_e), _shift_up(fd_e)
    h1e = (_dot(uo_m1, w1u_ref[0]) + _dot(fo_m1, w1f_ref[0])
           + _dot(up_e, w1u_ref[1]) + _dot(fd_e, w1f_ref[1])
           + _dot(up_o, w1u_ref[2]) + _dot(fd_o, w1f_ref[2]))
    h1o = (_dot(up_e, w1u_ref[0]) + _dot(fd_e, w1f_ref[0])
           + _dot(up_o, w1u_ref[1]) + _dot(fd_o, w1f_ref[1])
           + _dot(ue_p1, w1u_ref[2]) + _dot(fe_p1, w1f_ref[2]))
    h1e = jnp.maximum(h1e + b1r_ref[...], 0.0).astype(bf16)
    h1o = jnp.maximum(h1o + b1r_ref[...], 0.0).astype(bf16)

    ho_m1 = _shift_down(h1o)
    he_p1 = _shift_up(h1e)
    oe = (_dot(ho_m1, w2_ref[0]) + _dot(h1e, w2_ref[1])
          + _dot(h1o, w2_ref[2]))
    oo = (_dot(h1e, w2_ref[0]) + _dot(h1o, w2_ref[1])
          + _dot(he_p1, w2_ref[2]))
    o_ref[0, :, 0, :] = jnp.maximum(oe + b2r_ref[...], 0.0)
    o_ref[0, :, 1, :] = jnp.maximum(oo + b2r_ref[...], 0.0)


def kernel(from_down, from_up, wt, bt, w1, b1, w2, b2):
    N, Cout, Hd, Wd = from_down.shape
    _, Cin, Hu, Wu = from_up.shape
    bf16 = jnp.bfloat16
    Ku = Wu * Cin
    Nw = Wd * Cout

    fu = jnp.transpose(from_up, (0, 2, 3, 1)).reshape(N, Hu, Ku).astype(bf16)
    fd = jnp.transpose(from_down, (0, 2, 3, 1)).reshape(N, Hu, 2, Nw).astype(bf16)

    mu = _up_pair_mats(wt, Wu).astype(bf16)
    w1u = _band_mats(w1[:, :Cout], Wd).astype(bf16)
    w1f = _band_mats(w1[:, Cout:], Wd).astype(bf16)
    w2b = _band_mats(w2, Wd).astype(bf16)
    btr = jnp.tile(bt.astype(jnp.float32), Wd).reshape(1, Nw)
    b1r = jnp.tile(b1.astype(jnp.float32), Wd).reshape(1, Nw)
    b2r = jnp.tile(b2.astype(jnp.float32), Wd).reshape(1, Nw)

    out = pl.pallas_call(
        _fused_kernel,
        out_shape=jax.ShapeDtypeStruct((N, Hu, 2, Nw), jnp.float32),
        grid=(N,),
        in_specs=[
            pl.BlockSpec((1, Hu, Ku), lambda n: (n, 0, 0)),
            pl.BlockSpec((1, Hu, 2, Nw), lambda n: (n, 0, 0, 0)),
            pl.BlockSpec((2, Ku, Nw), lambda n: (0, 0, 0)),
            pl.BlockSpec((3, Nw, Nw), lambda n: (0, 0, 0)),
            pl.BlockSpec((3, Nw, Nw), lambda n: (0, 0, 0)),
            pl.BlockSpec((3, Nw, Nw), lambda n: (0, 0, 0)),
            pl.BlockSpec((1, Nw), lambda n: (0, 0)),
            pl.BlockSpec((1, Nw), lambda n: (0, 0)),
            pl.BlockSpec((1, Nw), lambda n: (0, 0)),
        ],
        out_specs=pl.BlockSpec((1, Hu, 2, Nw), lambda n: (n, 0, 0, 0)),
        compiler_params=pltpu.CompilerParams(
            dimension_semantics=("parallel",),
            vmem_limit_bytes=64 * 1024 * 1024,
        ),
    )(fu, fd, mu, w1u, w1f, w2b, btr, b1r, b2r)

    out = out.reshape(N, Hd, Wd, Cout)
    return jnp.transpose(out, (0, 3, 1, 2))


# speedup vs baseline: 1.0866x; 1.0866x over previous
import functools

import numpy as np
import jax
import jax.numpy as jnp
from jax.experimental import pallas as pl
from jax.experimental.pallas import tpu as pltpu


def _up_sel(Wu):
    Wd = 2 * Wu
    sel = np.zeros((Wu, Wd, 2), np.float32)
    for w in range(Wu):
        for dj in range(2):
            sel[w, 2 * w + dj, dj] = 1.0
    return sel


def _band_sel(Wd):
    sel = np.zeros((Wd, Wd, 3), np.float32)
    for i in range(Wd):
        for dx in range(3):
            j = i - (dx - 1)
            if 0 <= j < Wd:
                sel[i, j, dx] = 1.0
    return sel


def _up_pair_mats(wt, Wu):
    Cin, Cout = wt.shape[0], wt.shape[1]
    Wd = 2 * Wu
    m = jnp.einsum('wjd,copd->pojcw', _up_sel(Wu), wt.astype(jnp.float32))
    return m.reshape(2, Cout * Wd, Cin * Wu)


def _band_mats(w_oihw, Wd):
    m = jnp.einsum('ijd,ockd->kojci', _band_sel(Wd), w_oihw.astype(jnp.float32))
    Cout, Cin = w_oihw.shape[0], w_oihw.shape[1]
    return m.reshape(3, Cout * Wd, Cin * Wd)


def _dot(a, b):
    return jnp.dot(a, b, preferred_element_type=jnp.float32)


def _cshift_m1(x):
    return jnp.concatenate([jnp.zeros_like(x[:, :1]), x[:, :-1]], axis=1)


def _cshift_p1(x):
    return jnp.concatenate([x[:, 1:], jnp.zeros_like(x[:, :1])], axis=1)


def _fused_kernel(Hu, Wu, Cin, Cout,
                  fu_ref, fd_ref, mu_ref, w1u_ref, w1f_ref, w2_ref,
                  btc_ref, b1c_ref, b2c_ref, o_ref):
    bf16 = jnp.bfloat16
    Wd = 2 * Wu
    Nw = Cout * Wd

    fu = fu_ref[0].astype(bf16)
    xfu = jnp.transpose(fu, (0, 2, 1)).reshape(Cin * Wu, Hu)
    fde = fd_ref[0, :, :, 0, :].astype(bf16)
    fdo = fd_ref[0, :, :, 1, :].astype(bf16)
    fd_e = jnp.transpose(fde, (0, 2, 1)).reshape(Nw, Hu)
    fd_o = jnp.transpose(fdo, (0, 2, 1)).reshape(Nw, Hu)

    up_e = (_dot(mu_ref[0], xfu) + btc_ref[...]).astype(bf16)
    up_o = (_dot(mu_ref[1], xfu) + btc_ref[...]).astype(bf16)

    uo_m1, fo_m1 = _cshift_m1(up_o), _cshift_m1(fd_o)
    ue_p1, fe_p1 = _cshift_p1(up_e), _cshift_p1(fd_e)
    h1e = (_dot(w1u_ref[0], uo_m1) + _dot(w1f_ref[0], fo_m1)
           + _dot(w1u_ref[1], up_e) + _dot(w1f_ref[1], fd_e)
           + _dot(w1u_ref[2], up_o) + _dot(w1f_ref[2], fd_o))
    h1o = (_dot(w1u_ref[0], up_e) + _dot(w1f_ref[0], fd_e)
           + _dot(w1u_ref[1], up_o) + _dot(w1f_ref[1], fd_o)
           + _dot(w1u_ref[2], ue_p1) + _dot(w1f_ref[2], fe_p1))
    h1e = jnp.maximum(h1e + b1c_ref[...], 0.0).astype(bf16)
    h1o = jnp.maximum(h1o + b1c_ref[...], 0.0).astype(bf16)

    ho_m1 = _cshift_m1(h1o)
    he_p1 = _cshift_p1(h1e)
    oe = _dot(w2_ref[0], ho_m1) + _dot(w2_ref[1], h1e) + _dot(w2_ref[2], h1o)
    oo = _dot(w2_ref[0], h1e) + _dot(w2_ref[1], h1o) + _dot(w2_ref[2], he_p1)
    oe = jnp.maximum(oe + b2c_ref[...], 0.0)
    oo = jnp.maximum(oo + b2c_ref[...], 0.0)

    oe3 = jnp.transpose(oe.reshape(Cout, Wd, Hu), (0, 2, 1))
    oo3 = jnp.transpose(oo.reshape(Cout, Wd, Hu), (0, 2, 1))
    o_ref[0, :, :, 0, :] = oe3
    o_ref[0, :, :, 1, :] = oo3


def kernel(from_down, from_up, wt, bt, w1, b1, w2, b2):
    N, Cout, Hd, Wd = from_down.shape
    _, Cin, Hu, Wu = from_up.shape
    bf16 = jnp.bfloat16
    Ku = Cin * Wu
    Nw = Cout * Wd

    fd = from_down.reshape(N, Cout, Hu, 2, Wd)

    mu = _up_pair_mats(wt, Wu).astype(bf16)
    w1u = _band_mats(w1[:, :Cout], Wd).astype(bf16)
    w1f = _band_mats(w1[:, Cout:], Wd).astype(bf16)
    w2b = _band_mats(w2, Wd).astype(bf16)
    btc = jnp.repeat(bt.astype(jnp.float32), Wd).reshape(Nw, 1)
    b1c = jnp.repeat(b1.astype(jnp.float32), Wd).reshape(Nw, 1)
    b2c = jnp.repeat(b2.astype(jnp.float32), Wd).reshape(Nw, 1)

    out = pl.pallas_call(
        functools.partial(_fused_kernel, Hu, Wu, Cin, Cout),
        out_shape=jax.ShapeDtypeStruct((N, Cout, Hu, 2, Wd), jnp.float32),
        grid=(N,),
        in_specs=[
            pl.BlockSpec((1, Cin, Hu, Wu), lambda n: (n, 0, 0, 0)),
            pl.BlockSpec((1, Cout, Hu, 2, Wd), lambda n: (n, 0, 0, 0, 0)),
            pl.BlockSpec((2, Nw, Ku), lambda n: (0, 0, 0)),
            pl.BlockSpec((3, Nw, Nw), lambda n: (0, 0, 0)),
            pl.BlockSpec((3, Nw, Nw), lambda n: (0, 0, 0)),
            pl.BlockSpec((3, Nw, Nw), lambda n: (0, 0, 0)),
            pl.BlockSpec((Nw, 1), lambda n: (0, 0)),
            pl.BlockSpec((Nw, 1), lambda n: (0, 0)),
            pl.BlockSpec((Nw, 1), lambda n: (0, 0)),
        ],
        out_specs=pl.BlockSpec((1, Cout, Hu, 2, Wd), lambda n: (n, 0, 0, 0, 0)),
        compiler_params=pltpu.CompilerParams(
            dimension_semantics=("parallel",),
            vmem_limit_bytes=64 * 1024 * 1024,
        ),
    )(from_up, fd, mu, w1u, w1f, w2b, btc, b1c, b2c)

    return out.reshape(N, Cout, Hd, Wd)
